# Initial kernel scaffold; baseline (speedup 1.0000x reference)
import jax
import jax.numpy as jnp
from jax.experimental import pallas as pl


def kernel(edge_costs, t12_costs, t13_costs, t23_costs, tri_corr_12, tri_corr_13, tri_corr_23, edge_counter, W1, b1, W2, b2, W3, b3, W4, b4, W5, b5, W6, b6, g1, be1, g2, be2, g3, be3):
    raise NotImplementedError("write your pallas kernel here")


# speedup vs baseline: 2.3775x; 2.3775x over previous
import functools

import jax
import jax.numpy as jnp
from jax import lax
from jax.experimental import pallas as pl
from jax.experimental.pallas import tpu as pltpu
from jax.experimental.pallas import tpu_sc as plsc

E = 1_600_000
N = 1_600_000
NP = N // 8
R = 12_800
RP = R // 8
GRID = N // R

NC = 2
NS = 16
NW = NC * NS
PW = N // NW
CH = 10_000
SEG = E // NS

_f32 = jnp.float32


def _make_sc_gather():
    mesh = plsc.VectorSubcoreMesh(
        core_axis_name="c", subcore_axis_name="s", num_cores=NC,
        num_subcores=NS)

    @functools.partial(
        pl.kernel, mesh=mesh,
        out_type=[jax.ShapeDtypeStruct((N,), _f32) for _ in range(3)],
        scratch_types=[
            pltpu.VMEM((CH,), jnp.int32),
            pltpu.VMEM((CH,), _f32),
            pltpu.SemaphoreType.DMA,
        ],
    )
    def gather_k(q_hbm, i12, i13, i23, g12, g13, g23, idx_v, val_v, sem):
        wid = lax.axis_index("s") * NC + lax.axis_index("c")
        base = pl.multiple_of(wid * PW, 8)
        for ih, gh in ((i12, g12), (i13, g13), (i23, g23)):
            for c in range(PW // CH):
                off = pl.multiple_of(base + c * CH, 8)
                pltpu.sync_copy(ih.at[pl.ds(off, CH)], idx_v)
                pltpu.async_copy(q_hbm.at[idx_v], val_v, sem).wait()
                pltpu.sync_copy(val_v, gh.at[pl.ds(off, CH)])

    return gather_k


def _make_sc_scatter():
    mesh = plsc.VectorSubcoreMesh(
        core_axis_name="c", subcore_axis_name="s", num_cores=NC,
        num_subcores=NS)

    @functools.partial(
        pl.kernel, mesh=mesh,
        out_type=jax.ShapeDtypeStruct((NC * E,), _f32),
        scratch_types=[
            pltpu.VMEM((CH,), jnp.int32),
            pltpu.VMEM((CH,), _f32),
            pltpu.VMEM_SHARED((E,), _f32),
        ],
    )
    def scatter_k(d12, d13, d23, i12, i13, i23, zeros_hbm, out,
                  idx_v, val_v, acc):
        cid = lax.axis_index("c")
        sid = lax.axis_index("s")
        wid = sid * NC + cid
        soff = pl.multiple_of(sid * SEG, 8)
        for c in range(SEG // CH):
            zoff = pl.multiple_of(soff + c * CH, 8)
            pltpu.sync_copy(zeros_hbm.at[pl.ds(zoff, CH)], val_v)
            pltpu.sync_copy(val_v, acc.at[pl.ds(zoff, CH)])
        plsc.subcore_barrier()
        base = pl.multiple_of(wid * PW, 8)
        for dh, ih in ((d12, i12), (d13, i13), (d23, i23)):
            for c in range(PW // CH):
                off = pl.multiple_of(base + c * CH, 8)
                pltpu.sync_copy(ih.at[pl.ds(off, CH)], idx_v)
                pltpu.sync_copy(dh.at[pl.ds(off, CH)], val_v)
                pltpu.sync_copy(val_v, acc.at[idx_v], add=True)
        plsc.subcore_barrier()
        for c in range(SEG // CH):
            aoff = pl.multiple_of(soff + c * CH, 8)
            ooff = pl.multiple_of(cid * E + soff + c * CH, 8)
            pltpu.sync_copy(acc.at[pl.ds(aoff, CH)], val_v)
            pltpu.sync_copy(val_v, out.at[pl.ds(ooff, CH)])

    return scatter_k


_sc_gather = _make_sc_gather()
_sc_scatter = _make_sc_scatter()


def _relu(x):
    return jnp.maximum(x, 0.0)


def _prep_body(ec_ref, ct_ref, q_ref, ecm_ref):
    ec = ec_ref[...]
    ct = ct_ref[...]
    q_ref[...] = ec / ct
    ecm_ref[...] = jnp.where(ct > 0, jnp.zeros_like(ec), ec)


def _merge_body(p0_ref, p1_ref, ecm_ref, out_ref):
    out_ref[...] = p0_ref[...] + p1_ref[...] + ecm_ref[...]


def _stats_acc(st_ref, h, width):
    s = jnp.sum(h, axis=0)
    sq = jnp.sum(h * h, axis=0)
    acc = jnp.concatenate(
        [s[None], sq[None], jnp.zeros((6, width), _f32)], axis=0)

    @pl.when(pl.program_id(0) == 0)
    def _():
        st_ref[...] = acc

    @pl.when(pl.program_id(0) > 0)
    def _():
        st_ref[...] += acc


def _pass1_body(g12, g13, g23, t12, t13, t23, w1, b1, w2, b2, x_out, st_out):
    x = jnp.concatenate(
        [g12[...] + t12[...], g13[...] + t13[...], g23[...] + t23[...]],
        axis=1)
    x_out[...] = x
    h = _relu(jnp.dot(x, w1[...], preferred_element_type=_f32, precision=jax.lax.Precision.HIGHEST) + b1[...])
    h = _relu(jnp.dot(h, w2[...], preferred_element_type=_f32, precision=jax.lax.Precision.HIGHEST) + b2[...])
    _stats_acc(st_out, h, 256)


def _pass2_body(x, w1, b1, w2, b2, w3, b3, st_out):
    h = _relu(jnp.dot(x[...], w1[...], preferred_element_type=_f32, precision=jax.lax.Precision.HIGHEST) + b1[...])
    h = _relu(jnp.dot(h, w2[...], preferred_element_type=_f32, precision=jax.lax.Precision.HIGHEST) + b2[...])
    h = _relu(jnp.dot(h, w3[...], preferred_element_type=_f32, precision=jax.lax.Precision.HIGHEST) + b3[...])
    _stats_acc(st_out, h, 512)


def _pass3_body(x, w1, b1, w2, b2, w3, b3, w4, b4, st_out):
    h = _relu(jnp.dot(x[...], w1[...], preferred_element_type=_f32, precision=jax.lax.Precision.HIGHEST) + b1[...])
    h = _relu(jnp.dot(h, w2[...], preferred_element_type=_f32, precision=jax.lax.Precision.HIGHEST) + b2[...])
    h = _relu(jnp.dot(h, w3[...], preferred_element_type=_f32, precision=jax.lax.Precision.HIGHEST) + b3[...])
    h = _relu(jnp.dot(h, w4[...], preferred_element_type=_f32, precision=jax.lax.Precision.HIGHEST) + b4[...])
    _stats_acc(st_out, h, 256)


def _pass4_body(x, w1, b1, w2, b2, w3, b3, w4, b4, w5, b5, w6, b6,
                d12, d13, d23, t12o, t13o, t23o):
    xx = x[...]
    h = _relu(jnp.dot(xx, w1[...], preferred_element_type=_f32, precision=jax.lax.Precision.HIGHEST) + b1[...])
    h = _relu(jnp.dot(h, w2[...], preferred_element_type=_f32, precision=jax.lax.Precision.HIGHEST) + b2[...])
    h = _relu(jnp.dot(h, w3[...], preferred_element_type=_f32, precision=jax.lax.Precision.HIGHEST) + b3[...])
    h = _relu(jnp.dot(h, w4[...], preferred_element_type=_f32, precision=jax.lax.Precision.HIGHEST) + b4[...])
    h = _relu(jnp.dot(h, w5[...], preferred_element_type=_f32, precision=jax.lax.Precision.HIGHEST) + b5[...])
    d = jnp.dot(h, w6[...], preferred_element_type=_f32, precision=jax.lax.Precision.HIGHEST) + b6[...]
    d12[...] = d[:, 0:8]
    d13[...] = d[:, 8:16]
    d23[...] = d[:, 16:24]
    t12o[...] = xx[:, 0:8] - d[:, 0:8]
    t13o[...] = xx[:, 8:16] - d[:, 8:16]
    t23o[...] = xx[:, 16:24] - d[:, 16:24]


def _row_spec(width):
    return pl.BlockSpec((RP, width), lambda i: (i, 0))


def _wspec():
    return pl.BlockSpec(index_map=lambda i: (0, 0))


_seq = pltpu.CompilerParams(dimension_semantics=("arbitrary",))


def kernel(edge_costs, t12_costs, t13_costs, t23_costs, tri_corr_12,
           tri_corr_13, tri_corr_23, edge_counter, W1, b1, W2, b2, W3, b3,
           W4, b4, W5, b5, W6, b6, g1, be1, g2, be2, g3, be3):
    ec2 = edge_costs.reshape(3125, 512)
    ct2 = edge_counter.reshape(3125, 512)
    q2, ecm2 = pl.pallas_call(
        _prep_body,
        out_shape=[jax.ShapeDtypeStruct((3125, 512), _f32)] * 2,
    )(ec2, ct2)

    g12, g13, g23 = _sc_gather(
        q2.reshape(E), tri_corr_12, tri_corr_13, tri_corr_23)

    eye8 = jnp.eye(8, dtype=_f32)
    ones8 = jnp.ones((8,), _f32)

    def pw(W):
        return jnp.kron(W.T.astype(_f32), eye8)

    def pv(v):
        return jnp.kron(v.astype(_f32), ones8)

    W1p, W2p, W3p, W4p, W5p, W6p = pw(W1), pw(W2), pw(W3), pw(W4), pw(W5), pw(W6)
    b1p, b2p, b6p = pv(b1)[None], pv(b2)[None], pv(b6)[None]

    def fold(stats, width, feats, g, be, Wnext_p, bnext):
        s = stats[0].reshape(feats, 8).sum(1)
        sq = stats[1].reshape(feats, 8).sum(1)
        m = s / N
        v = sq / N - m * m
        a = g / jnp.sqrt(v + 1e-5)
        c = be - m * a
        ap = pv(a)
        cp = pv(c)
        Wf = ap[:, None] * Wnext_p
        bf = (pv(bnext) + cp @ Wnext_p)[None]
        return Wf, bf

    gs = [a.reshape(NP, 8) for a in (g12, g13, g23)]
    ts = [a.reshape(NP, 8) for a in (t12_costs, t13_costs, t23_costs)]

    x_p, st1 = pl.pallas_call(
        _pass1_body,
        grid=(GRID,),
        in_specs=[_row_spec(8)] * 6 + [_wspec()] * 4,
        out_specs=[_row_spec(24), pl.BlockSpec((8, 256), lambda i: (0, 0))],
        out_shape=[jax.ShapeDtypeStruct((NP, 24), _f32),
                   jax.ShapeDtypeStruct((8, 256), _f32)],
        compiler_params=_seq,
    )(*gs, *ts, W1p, b1p, W2p, b2p)

    W3f, b3f = fold(st1, 256, 32, g1, be1, W3p, b3)

    st2 = pl.pallas_call(
        _pass2_body,
        grid=(GRID,),
        in_specs=[_row_spec(24)] + [_wspec()] * 6,
        out_specs=pl.BlockSpec((8, 512), lambda i: (0, 0)),
        out_shape=jax.ShapeDtypeStruct((8, 512), _f32),
        compiler_params=_seq,
    )(x_p, W1p, b1p, W2p, b2p, W3f, b3f)

    W4f, b4f = fold(st2, 512, 64, g2, be2, W4p, b4)

    st3 = pl.pallas_call(
        _pass3_body,
        grid=(GRID,),
        in_specs=[_row_spec(24)] + [_wspec()] * 8,
        out_specs=pl.BlockSpec((8, 256), lambda i: (0, 0)),
        out_shape=jax.ShapeDtypeStruct((8, 256), _f32),
        compiler_params=_seq,
    )(x_p, W1p, b1p, W2p, b2p, W3f, b3f, W4f, b4f)

    W5f, b5f = fold(st3, 256, 32, g3, be3, W5p, b5)

    d12, d13, d23, t12o, t13o, t23o = pl.pallas_call(
        _pass4_body,
        grid=(GRID,),
        in_specs=[_row_spec(24)] + [_wspec()] * 12,
        out_specs=[_row_spec(8)] * 6,
        out_shape=[jax.ShapeDtypeStruct((NP, 8), _f32)] * 6,
        compiler_params=_seq,
    )(x_p, W1p, b1p, W2p, b2p, W3f, b3f, W4f, b4f, W5f, b5f, W6p, b6p)

    zeros = jnp.zeros((E,), _f32)
    parts = _sc_scatter(
        d12.reshape(N), d13.reshape(N), d23.reshape(N),
        tri_corr_12, tri_corr_13, tri_corr_23, zeros)

    ec_out = pl.pallas_call(
        _merge_body,
        out_shape=jax.ShapeDtypeStruct((3125, 512), _f32),
    )(parts[:E].reshape(3125, 512), parts[E:].reshape(3125, 512), ecm2)

    return (ec_out.reshape(E), t12o.reshape(N), t13o.reshape(N),
            t23o.reshape(N))


# speedup vs baseline: 3.9073x; 1.6434x over previous
import functools

import jax
import jax.numpy as jnp
from jax import lax
from jax.experimental import pallas as pl
from jax.experimental.pallas import tpu as pltpu
from jax.experimental.pallas import tpu_sc as plsc

E = 1_600_000
N = 1_600_000
NP = N // 8
R = 12_800
RP = R // 8
GRID = N // R

NC = 2
NS = 16
NW = NC * NS
PW = N // NW
CH = 10_000
SEG = E // NS

_f32 = jnp.float32


def _make_sc_gather():
    mesh = plsc.VectorSubcoreMesh(
        core_axis_name="c", subcore_axis_name="s", num_cores=NC,
        num_subcores=NS)

    @functools.partial(
        pl.kernel, mesh=mesh,
        out_type=[jax.ShapeDtypeStruct((N,), _f32) for _ in range(3)],
        scratch_types=[
            pltpu.VMEM((CH,), jnp.int32),
            pltpu.VMEM((CH,), _f32),
            pltpu.SemaphoreType.DMA,
        ],
    )
    def gather_k(q_hbm, i12, i13, i23, g12, g13, g23, idx_v, val_v, sem):
        wid = lax.axis_index("s") * NC + lax.axis_index("c")
        base = pl.multiple_of(wid * PW, 8)
        for ih, gh in ((i12, g12), (i13, g13), (i23, g23)):
            for c in range(PW // CH):
                off = pl.multiple_of(base + c * CH, 8)
                pltpu.sync_copy(ih.at[pl.ds(off, CH)], idx_v)
                pltpu.async_copy(q_hbm.at[idx_v], val_v, sem).wait()
                pltpu.sync_copy(val_v, gh.at[pl.ds(off, CH)])

    return gather_k


def _make_sc_scatter():
    mesh = plsc.VectorSubcoreMesh(
        core_axis_name="c", subcore_axis_name="s", num_cores=NC,
        num_subcores=NS)

    @functools.partial(
        pl.kernel, mesh=mesh,
        out_type=jax.ShapeDtypeStruct((NC * E,), _f32),
        scratch_types=[
            pltpu.VMEM((CH,), jnp.int32),
            pltpu.VMEM((CH,), _f32),
            pltpu.VMEM_SHARED((E,), _f32),
        ],
    )
    def scatter_k(d12, d13, d23, i12, i13, i23, zeros_hbm, out,
                  idx_v, val_v, acc):
        cid = lax.axis_index("c")
        sid = lax.axis_index("s")
        wid = sid * NC + cid
        soff = pl.multiple_of(sid * SEG, 8)
        for c in range(SEG // CH):
            zoff = pl.multiple_of(soff + c * CH, 8)
            pltpu.sync_copy(zeros_hbm.at[pl.ds(zoff, CH)], val_v)
            pltpu.sync_copy(val_v, acc.at[pl.ds(zoff, CH)])
        plsc.subcore_barrier()
        base = pl.multiple_of(wid * PW, 8)
        for dh, ih in ((d12, i12), (d13, i13), (d23, i23)):
            for c in range(PW // CH):
                off = pl.multiple_of(base + c * CH, 8)
                pltpu.sync_copy(ih.at[pl.ds(off, CH)], idx_v)
                pltpu.sync_copy(dh.at[pl.ds(off, CH)], val_v)
                pltpu.sync_copy(val_v, acc.at[idx_v], add=True)
        plsc.subcore_barrier()
        for c in range(SEG // CH):
            aoff = pl.multiple_of(soff + c * CH, 8)
            ooff = pl.multiple_of(cid * E + soff + c * CH, 8)
            pltpu.sync_copy(acc.at[pl.ds(aoff, CH)], val_v)
            pltpu.sync_copy(val_v, out.at[pl.ds(ooff, CH)])

    return scatter_k


_sc_gather = _make_sc_gather()
_sc_scatter = _make_sc_scatter()


def _relu(x):
    return jnp.maximum(x, 0.0)


def _prep_body(ec_ref, ct_ref, q_ref, ecm_ref):
    ec = ec_ref[...]
    ct = ct_ref[...]
    q_ref[...] = ec / ct
    ecm_ref[...] = jnp.where(ct > 0, jnp.zeros_like(ec), ec)


def _merge_body(p0_ref, p1_ref, ecm_ref, out_ref):
    out_ref[...] = p0_ref[...] + p1_ref[...] + ecm_ref[...]


def _stats_acc(st_ref, h, width):
    s = jnp.sum(h, axis=0)
    sq = jnp.sum(h * h, axis=0)
    acc = jnp.concatenate(
        [s[None], sq[None], jnp.zeros((6, width), _f32)], axis=0)

    @pl.when(pl.program_id(0) == 0)
    def _():
        st_ref[...] = acc

    @pl.when(pl.program_id(0) > 0)
    def _():
        st_ref[...] += acc


def _dot(a, b):
    return jnp.dot(a, b, preferred_element_type=_f32,
                   precision=jax.lax.Precision.HIGHEST)


def _pass1_body(g12, g13, g23, t12, t13, t23, w1, b1, w2, b2, h2_out, st_out):
    x = jnp.concatenate(
        [g12[...] + t12[...], g13[...] + t13[...], g23[...] + t23[...]],
        axis=1)
    h = _relu(_dot(x, w1[...]) + b1[...])
    h = _relu(_dot(h, w2[...]) + b2[...])
    h2_out[...] = h
    _stats_acc(st_out, h, 256)


def _pass2_body(h2, w3, b3, st_out):
    h = _relu(_dot(h2[...], w3[...]) + b3[...])
    _stats_acc(st_out, h, 512)


def _pass3_body(h2, w3, b3, w4, b4, h4_out, st_out):
    h = _relu(_dot(h2[...], w3[...]) + b3[...])
    h = _relu(_dot(h, w4[...]) + b4[...])
    h4_out[...] = h
    _stats_acc(st_out, h, 256)


def _pass4_body(h4, g12, g13, g23, t12, t13, t23, w5, b5, w6, b6,
                d12, d13, d23, t12o, t13o, t23o):
    h = _relu(_dot(h4[...], w5[...]) + b5[...])
    d = _dot(h, w6[...]) + b6[...]
    d12[...] = d[:, 0:8]
    d13[...] = d[:, 8:16]
    d23[...] = d[:, 16:24]
    t12o[...] = g12[...] + t12[...] - d[:, 0:8]
    t13o[...] = g13[...] + t13[...] - d[:, 8:16]
    t23o[...] = g23[...] + t23[...] - d[:, 16:24]


def _row_spec(width):
    return pl.BlockSpec((RP, width), lambda i: (i, 0))


def _wspec():
    return pl.BlockSpec(index_map=lambda i: (0, 0))


_seq = pltpu.CompilerParams(dimension_semantics=("arbitrary",))


def kernel(edge_costs, t12_costs, t13_costs, t23_costs, tri_corr_12,
           tri_corr_13, tri_corr_23, edge_counter, W1, b1, W2, b2, W3, b3,
           W4, b4, W5, b5, W6, b6, g1, be1, g2, be2, g3, be3):
    ec2 = edge_costs.reshape(3125, 512)
    ct2 = edge_counter.reshape(3125, 512)
    q2, ecm2 = pl.pallas_call(
        _prep_body,
        out_shape=[jax.ShapeDtypeStruct((3125, 512), _f32)] * 2,
    )(ec2, ct2)

    g12, g13, g23 = _sc_gather(
        q2.reshape(E), tri_corr_12, tri_corr_13, tri_corr_23)

    eye8 = jnp.eye(8, dtype=_f32)
    ones8 = jnp.ones((8,), _f32)

    def pw(W):
        return jnp.kron(W.T.astype(_f32), eye8)

    def pv(v):
        return jnp.kron(v.astype(_f32), ones8)

    W1p, W2p, W3p, W4p, W5p, W6p = pw(W1), pw(W2), pw(W3), pw(W4), pw(W5), pw(W6)
    b1p, b2p, b6p = pv(b1)[None], pv(b2)[None], pv(b6)[None]

    def fold(stats, width, feats, g, be, Wnext_p, bnext):
        s = stats[0].reshape(feats, 8).sum(1)
        sq = stats[1].reshape(feats, 8).sum(1)
        m = s / N
        v = sq / N - m * m
        a = g / jnp.sqrt(v + 1e-5)
        c = be - m * a
        ap = pv(a)
        cp = pv(c)
        Wf = ap[:, None] * Wnext_p
        bf = (pv(bnext) + cp @ Wnext_p)[None]
        return Wf, bf

    gs = [a.reshape(NP, 8) for a in (g12, g13, g23)]
    ts = [a.reshape(NP, 8) for a in (t12_costs, t13_costs, t23_costs)]

    h2_p, st1 = pl.pallas_call(
        _pass1_body,
        grid=(GRID,),
        in_specs=[_row_spec(8)] * 6 + [_wspec()] * 4,
        out_specs=[_row_spec(256), pl.BlockSpec((8, 256), lambda i: (0, 0))],
        out_shape=[jax.ShapeDtypeStruct((NP, 256), _f32),
                   jax.ShapeDtypeStruct((8, 256), _f32)],
        compiler_params=_seq,
    )(*gs, *ts, W1p, b1p, W2p, b2p)

    W3f, b3f = fold(st1, 256, 32, g1, be1, W3p, b3)

    st2 = pl.pallas_call(
        _pass2_body,
        grid=(GRID,),
        in_specs=[_row_spec(256)] + [_wspec()] * 2,
        out_specs=pl.BlockSpec((8, 512), lambda i: (0, 0)),
        out_shape=jax.ShapeDtypeStruct((8, 512), _f32),
        compiler_params=_seq,
    )(h2_p, W3f, b3f)

    W4f, b4f = fold(st2, 512, 64, g2, be2, W4p, b4)

    h4_p, st3 = pl.pallas_call(
        _pass3_body,
        grid=(GRID,),
        in_specs=[_row_spec(256)] + [_wspec()] * 4,
        out_specs=[_row_spec(256), pl.BlockSpec((8, 256), lambda i: (0, 0))],
        out_shape=[jax.ShapeDtypeStruct((NP, 256), _f32),
                   jax.ShapeDtypeStruct((8, 256), _f32)],
        compiler_params=_seq,
    )(h2_p, W3f, b3f, W4f, b4f)

    W5f, b5f = fold(st3, 256, 32, g3, be3, W5p, b5)

    d12, d13, d23, t12o, t13o, t23o = pl.pallas_call(
        _pass4_body,
        grid=(GRID,),
        in_specs=[_row_spec(256)] + [_row_spec(8)] * 6 + [_wspec()] * 4,
        out_specs=[_row_spec(8)] * 6,
        out_shape=[jax.ShapeDtypeStruct((NP, 8), _f32)] * 6,
        compiler_params=_seq,
    )(h4_p, *gs, *ts, W5f, b5f, W6p, b6p)

    zeros = jnp.zeros((E,), _f32)
    parts = _sc_scatter(
        d12.reshape(N), d13.reshape(N), d23.reshape(N),
        tri_corr_12, tri_corr_13, tri_corr_23, zeros)

    ec_out = pl.pallas_call(
        _merge_body,
        out_shape=jax.ShapeDtypeStruct((3125, 512), _f32),
    )(parts[:E].reshape(3125, 512), parts[E:].reshape(3125, 512), ecm2)

    return (ec_out.reshape(E), t12o.reshape(N), t13o.reshape(N),
            t23o.reshape(N))


# speedup vs baseline: 6.3331x; 1.6208x over previous
import functools

import jax
import jax.numpy as jnp
from jax import lax
from jax.experimental import pallas as pl
from jax.experimental.pallas import tpu as pltpu
from jax.experimental.pallas import tpu_sc as plsc

E = 1_600_000
N = 1_600_000
NP = N // 8
R = 12_800
RP = R // 8
GRID = N // R

NC = 2
NS = 16
NW = NC * NS
PW = N // NW
CH = 10_000
SEG = E // NS

_f32 = jnp.float32


def _make_sc_gather():
    mesh = plsc.VectorSubcoreMesh(
        core_axis_name="c", subcore_axis_name="s", num_cores=NC,
        num_subcores=NS)

    @functools.partial(
        pl.kernel, mesh=mesh,
        out_type=[jax.ShapeDtypeStruct((N,), _f32) for _ in range(3)],
        scratch_types=[
            pltpu.VMEM((CH,), jnp.int32),
            pltpu.VMEM((CH,), _f32),
            pltpu.SemaphoreType.DMA,
        ],
    )
    def gather_k(q_hbm, i12, i13, i23, g12, g13, g23, idx_v, val_v, sem):
        wid = lax.axis_index("s") * NC + lax.axis_index("c")
        base = pl.multiple_of(wid * PW, 8)
        for ih, gh in ((i12, g12), (i13, g13), (i23, g23)):
            for c in range(PW // CH):
                off = pl.multiple_of(base + c * CH, 8)
                pltpu.sync_copy(ih.at[pl.ds(off, CH)], idx_v)
                pltpu.async_copy(q_hbm.at[idx_v], val_v, sem).wait()
                pltpu.sync_copy(val_v, gh.at[pl.ds(off, CH)])

    return gather_k


def _make_sc_scatter():
    mesh = plsc.VectorSubcoreMesh(
        core_axis_name="c", subcore_axis_name="s", num_cores=NC,
        num_subcores=NS)

    @functools.partial(
        pl.kernel, mesh=mesh,
        out_type=jax.ShapeDtypeStruct((NC * E,), _f32),
        scratch_types=[
            pltpu.VMEM((CH,), jnp.int32),
            pltpu.VMEM((CH,), _f32),
            pltpu.VMEM_SHARED((E,), _f32),
        ],
    )
    def scatter_k(d12, d13, d23, i12, i13, i23, zeros_hbm, out,
                  idx_v, val_v, acc):
        cid = lax.axis_index("c")
        sid = lax.axis_index("s")
        wid = sid * NC + cid
        soff = pl.multiple_of(sid * SEG, 8)
        for c in range(SEG // CH):
            zoff = pl.multiple_of(soff + c * CH, 8)
            pltpu.sync_copy(zeros_hbm.at[pl.ds(zoff, CH)], val_v)
            pltpu.sync_copy(val_v, acc.at[pl.ds(zoff, CH)])
        plsc.subcore_barrier()
        base = pl.multiple_of(wid * PW, 8)
        for dh, ih in ((d12, i12), (d13, i13), (d23, i23)):
            for c in range(PW // CH):
                off = pl.multiple_of(base + c * CH, 8)
                pltpu.sync_copy(ih.at[pl.ds(off, CH)], idx_v)
                pltpu.sync_copy(dh.at[pl.ds(off, CH)], val_v)
                pltpu.sync_copy(val_v, acc.at[idx_v], add=True)
        plsc.subcore_barrier()
        for c in range(SEG // CH):
            aoff = pl.multiple_of(soff + c * CH, 8)
            ooff = pl.multiple_of(cid * E + soff + c * CH, 8)
            pltpu.sync_copy(acc.at[pl.ds(aoff, CH)], val_v)
            pltpu.sync_copy(val_v, out.at[pl.ds(ooff, CH)])

    return scatter_k


_sc_gather = _make_sc_gather()
_sc_scatter = _make_sc_scatter()


def _relu(x):
    return jnp.maximum(x, 0.0)


def _prep_body(ec_ref, ct_ref, q_ref, ecm_ref):
    ec = ec_ref[...]
    ct = ct_ref[...]
    q_ref[...] = ec / ct
    ecm_ref[...] = jnp.where(ct > 0, jnp.zeros_like(ec), ec)


def _merge_body(p0_ref, p1_ref, ecm_ref, out_ref):
    out_ref[...] = p0_ref[...] + p1_ref[...] + ecm_ref[...]


def _stats_acc(st_ref, h, width):
    s = jnp.sum(h, axis=0)
    sq = jnp.sum(h * h, axis=0)
    acc = jnp.concatenate(
        [s[None], sq[None], jnp.zeros((6, width), _f32)], axis=0)

    @pl.when(pl.program_id(0) == 0)
    def _():
        st_ref[...] = acc

    @pl.when(pl.program_id(0) > 0)
    def _():
        st_ref[...] += acc


_bf16 = jnp.bfloat16


def _dot(a, b):
    return jnp.dot(a.astype(_bf16), b, preferred_element_type=_f32)


def _pass1_body(g12, g13, g23, t12, t13, t23, w1, b1, w2, b2, h2_out, st_out):
    x = jnp.concatenate(
        [g12[...] + t12[...], g13[...] + t13[...], g23[...] + t23[...]],
        axis=1)
    h = _relu(_dot(x, w1[...]) + b1[...])
    h = _relu(_dot(h, w2[...]) + b2[...])
    h2_out[...] = h
    _stats_acc(st_out, h, 256)


def _bn(h, m, rg, be):
    return (h - m) * rg + be


def _pass2_body(h2, m1, rg1, be1, w3, b3, st_out):
    hn = _bn(h2[...], m1[...], rg1[...], be1[...])
    h = _relu(_dot(hn, w3[...]) + b3[...])
    _stats_acc(st_out, h, 512)


def _pass3_body(h2, m1, rg1, be1, w3, b3, m2, rg2, be2, w4, b4,
                h4_out, st_out):
    hn = _bn(h2[...], m1[...], rg1[...], be1[...])
    h = _relu(_dot(hn, w3[...]) + b3[...])
    hn = _bn(h, m2[...], rg2[...], be2[...])
    h = _relu(_dot(hn, w4[...]) + b4[...])
    h4_out[...] = h
    _stats_acc(st_out, h, 256)


def _pass4_body(h4, g12, g13, g23, t12, t13, t23, m3, rg3, be3, w5, b5,
                w6, b6, d12, d13, d23, t12o, t13o, t23o):
    hn = _bn(h4[...], m3[...], rg3[...], be3[...])
    h = _relu(_dot(hn, w5[...]) + b5[...])
    d = _dot(h, w6[...]) + b6[...]
    d12[...] = d[:, 0:8]
    d13[...] = d[:, 8:16]
    d23[...] = d[:, 16:24]
    t12o[...] = g12[...] + t12[...] - d[:, 0:8]
    t13o[...] = g13[...] + t13[...] - d[:, 8:16]
    t23o[...] = g23[...] + t23[...] - d[:, 16:24]


def _row_spec(width):
    return pl.BlockSpec((RP, width), lambda i: (i, 0))


def _wspec():
    return pl.BlockSpec(index_map=lambda i: (0, 0))


_seq = pltpu.CompilerParams(dimension_semantics=("arbitrary",))


def kernel(edge_costs, t12_costs, t13_costs, t23_costs, tri_corr_12,
           tri_corr_13, tri_corr_23, edge_counter, W1, b1, W2, b2, W3, b3,
           W4, b4, W5, b5, W6, b6, g1, be1, g2, be2, g3, be3):
    ec2 = edge_costs.reshape(3125, 512)
    ct2 = edge_counter.reshape(3125, 512)
    q2, ecm2 = pl.pallas_call(
        _prep_body,
        out_shape=[jax.ShapeDtypeStruct((3125, 512), _f32)] * 2,
    )(ec2, ct2)

    g12, g13, g23 = _sc_gather(
        q2.reshape(E), tri_corr_12, tri_corr_13, tri_corr_23)

    eye8 = jnp.eye(8, dtype=_f32)
    ones8 = jnp.ones((8,), _f32)

    def pw(W):
        return jnp.kron(W.T.astype(_f32), eye8)

    def pv(v):
        return jnp.kron(v.astype(_f32), ones8)

    W1p, W2p, W3p, W4p, W5p, W6p = (
        pw(W1).astype(_bf16), pw(W2).astype(_bf16), pw(W3).astype(_bf16),
        pw(W4).astype(_bf16), pw(W5).astype(_bf16), pw(W6).astype(_bf16))
    b1p, b2p, b3p, b4p, b5p, b6p = (
        pv(b1)[None], pv(b2)[None], pv(b3)[None], pv(b4)[None],
        pv(b5)[None], pv(b6)[None])

    def bn_params(stats, feats, g, be):
        s = stats[0].reshape(feats, 8).sum(1)
        sq = stats[1].reshape(feats, 8).sum(1)
        m = s / N
        v = sq / N - m * m
        rg = g / jnp.sqrt(v + 1e-5)
        return pv(m)[None], pv(rg)[None], pv(be)[None]

    gs = [a.reshape(NP, 8) for a in (g12, g13, g23)]
    ts = [a.reshape(NP, 8) for a in (t12_costs, t13_costs, t23_costs)]

    h2_p, st1 = pl.pallas_call(
        _pass1_body,
        grid=(GRID,),
        in_specs=[_row_spec(8)] * 6 + [_wspec()] * 4,
        out_specs=[_row_spec(256), pl.BlockSpec((8, 256), lambda i: (0, 0))],
        out_shape=[jax.ShapeDtypeStruct((NP, 256), _f32),
                   jax.ShapeDtypeStruct((8, 256), _f32)],
        compiler_params=_seq,
    )(*gs, *ts, W1p, b1p, W2p, b2p)

    m1p, rg1p, be1p = bn_params(st1, 32, g1, be1)

    st2 = pl.pallas_call(
        _pass2_body,
        grid=(GRID,),
        in_specs=[_row_spec(256)] + [_wspec()] * 5,
        out_specs=pl.BlockSpec((8, 512), lambda i: (0, 0)),
        out_shape=jax.ShapeDtypeStruct((8, 512), _f32),
        compiler_params=_seq,
    )(h2_p, m1p, rg1p, be1p, W3p, b3p)

    m2p, rg2p, be2p = bn_params(st2, 64, g2, be2)

    h4_p, st3 = pl.pallas_call(
        _pass3_body,
        grid=(GRID,),
        in_specs=[_row_spec(256)] + [_wspec()] * 10,
        out_specs=[_row_spec(256), pl.BlockSpec((8, 256), lambda i: (0, 0))],
        out_shape=[jax.ShapeDtypeStruct((NP, 256), _f32),
                   jax.ShapeDtypeStruct((8, 256), _f32)],
        compiler_params=_seq,
    )(h2_p, m1p, rg1p, be1p, W3p, b3p, m2p, rg2p, be2p, W4p, b4p)

    m3p, rg3p, be3p = bn_params(st3, 32, g3, be3)

    d12, d13, d23, t12o, t13o, t23o = pl.pallas_call(
        _pass4_body,
        grid=(GRID,),
        in_specs=[_row_spec(256)] + [_row_spec(8)] * 6 + [_wspec()] * 7,
        out_specs=[_row_spec(8)] * 6,
        out_shape=[jax.ShapeDtypeStruct((NP, 8), _f32)] * 6,
        compiler_params=_seq,
    )(h4_p, *gs, *ts, m3p, rg3p, be3p, W5p, b5p, W6p, b6p)

    zeros = jnp.zeros((E,), _f32)
    parts = _sc_scatter(
        d12.reshape(N), d13.reshape(N), d23.reshape(N),
        tri_corr_12, tri_corr_13, tri_corr_23, zeros)

    ec_out = pl.pallas_call(
        _merge_body,
        out_shape=jax.ShapeDtypeStruct((3125, 512), _f32),
    )(parts[:E].reshape(3125, 512), parts[E:].reshape(3125, 512), ecm2)

    return (ec_out.reshape(E), t12o.reshape(N), t13o.reshape(N),
            t23o.reshape(N))


# speedup vs baseline: 6.5864x; 1.0400x over previous
import functools

import jax
import jax.numpy as jnp
from jax import lax
from jax.experimental import pallas as pl
from jax.experimental.pallas import tpu as pltpu
from jax.experimental.pallas import tpu_sc as plsc

E = 1_600_000
N = 1_600_000
NP = N // 8
R = 12_800
RP = R // 8
GRID = N // R

NC = 2
NS = 16
NW = NC * NS
PW = N // NW
CH = 10_000
SEG = E // NS

_f32 = jnp.float32


def _make_sc_gather():
    mesh = plsc.VectorSubcoreMesh(
        core_axis_name="c", subcore_axis_name="s", num_cores=NC,
        num_subcores=NS)

    @functools.partial(
        pl.kernel, mesh=mesh,
        out_type=[jax.ShapeDtypeStruct((N,), _f32) for _ in range(3)],
        scratch_types=[
            pltpu.VMEM((CH,), jnp.int32),
            pltpu.VMEM((CH,), _f32),
            pltpu.SemaphoreType.DMA,
        ],
    )
    def gather_k(q_hbm, i12, i13, i23, g12, g13, g23, idx_v, val_v, sem):
        wid = lax.axis_index("s") * NC + lax.axis_index("c")
        base = pl.multiple_of(wid * PW, 8)
        for ih, gh in ((i12, g12), (i13, g13), (i23, g23)):
            for c in range(PW // CH):
                off = pl.multiple_of(base + c * CH, 8)
                pltpu.sync_copy(ih.at[pl.ds(off, CH)], idx_v)
                pltpu.async_copy(q_hbm.at[idx_v], val_v, sem).wait()
                pltpu.sync_copy(val_v, gh.at[pl.ds(off, CH)])

    return gather_k


def _make_sc_scatter():
    mesh = plsc.VectorSubcoreMesh(
        core_axis_name="c", subcore_axis_name="s", num_cores=NC,
        num_subcores=NS)

    @functools.partial(
        pl.kernel, mesh=mesh,
        out_type=jax.ShapeDtypeStruct((NC * E,), _f32),
        scratch_types=[
            pltpu.VMEM((CH,), jnp.int32),
            pltpu.VMEM((CH,), _f32),
            pltpu.VMEM_SHARED((E,), _f32),
        ],
    )
    def scatter_k(d12, d13, d23, i12, i13, i23, zeros_hbm, out,
                  idx_v, val_v, acc):
        cid = lax.axis_index("c")
        sid = lax.axis_index("s")
        wid = sid * NC + cid
        soff = pl.multiple_of(sid * SEG, 8)
        for c in range(SEG // CH):
            zoff = pl.multiple_of(soff + c * CH, 8)
            pltpu.sync_copy(zeros_hbm.at[pl.ds(zoff, CH)], val_v)
            pltpu.sync_copy(val_v, acc.at[pl.ds(zoff, CH)])
        plsc.subcore_barrier()
        base = pl.multiple_of(wid * PW, 8)
        for dh, ih in ((d12, i12), (d13, i13), (d23, i23)):
            for c in range(PW // CH):
                off = pl.multiple_of(base + c * CH, 8)
                pltpu.sync_copy(ih.at[pl.ds(off, CH)], idx_v)
                pltpu.sync_copy(dh.at[pl.ds(off, CH)], val_v)
                pltpu.sync_copy(val_v, acc.at[idx_v], add=True)
        plsc.subcore_barrier()
        for c in range(SEG // CH):
            aoff = pl.multiple_of(soff + c * CH, 8)
            ooff = pl.multiple_of(cid * E + soff + c * CH, 8)
            pltpu.sync_copy(acc.at[pl.ds(aoff, CH)], val_v)
            pltpu.sync_copy(val_v, out.at[pl.ds(ooff, CH)])

    return scatter_k


_sc_gather = _make_sc_gather()
_sc_scatter = _make_sc_scatter()


def _relu(x):
    return jnp.maximum(x, 0.0)


def _prep_body(ec_ref, ct_ref, q_ref, ecm_ref):
    ec = ec_ref[...]
    ct = ct_ref[...]
    q_ref[...] = ec / ct
    ecm_ref[...] = jnp.where(ct > 0, jnp.zeros_like(ec), ec)


def _merge_body(p0_ref, p1_ref, ecm_ref, out_ref):
    out_ref[...] = p0_ref[...] + p1_ref[...] + ecm_ref[...]


def _stats_acc(st_ref, h, width):
    s = jnp.sum(h, axis=0)
    sq = jnp.sum(h * h, axis=0)
    acc = jnp.concatenate(
        [s[None], sq[None], jnp.zeros((6, width), _f32)], axis=0)

    @pl.when(pl.program_id(0) == 0)
    def _():
        st_ref[...] = acc

    @pl.when(pl.program_id(0) > 0)
    def _():
        st_ref[...] += acc


_bf16 = jnp.bfloat16


def _dot(a, b):
    return jnp.dot(a.astype(_bf16), b, preferred_element_type=_f32)


def _bn(h, m, rg, be):
    return (h - m) * rg + be


def _to_h2(x, w1, b1, w2, b2):
    h = _relu(_dot(x, w1[...]) + b1[...])
    return _relu(_dot(h, w2[...]) + b2[...])


def _to_h3(h2, m1, rg1, be1, w3, b3):
    hn = _bn(h2, m1[...], rg1[...], be1[...])
    return _relu(_dot(hn, w3[...]) + b3[...])


def _to_h4(h3, m2, rg2, be2, w4, b4):
    hn = _bn(h3, m2[...], rg2[...], be2[...])
    return _relu(_dot(hn, w4[...]) + b4[...])


def _pass1_body(g12, g13, g23, t12, t13, t23, w1, b1, w2, b2, x_out, st_out):
    x = jnp.concatenate(
        [g12[...] + t12[...], g13[...] + t13[...], g23[...] + t23[...]],
        axis=1)
    x_out[...] = x
    h = _to_h2(x, w1, b1, w2, b2)
    _stats_acc(st_out, h, 256)


def _pass2_body(x, w1, b1, w2, b2, m1, rg1, be1, w3, b3, st_out):
    h = _to_h2(x[...], w1, b1, w2, b2)
    h = _to_h3(h, m1, rg1, be1, w3, b3)
    _stats_acc(st_out, h, 512)


def _pass3_body(x, w1, b1, w2, b2, m1, rg1, be1, w3, b3, m2, rg2, be2,
                w4, b4, st_out):
    h = _to_h2(x[...], w1, b1, w2, b2)
    h = _to_h3(h, m1, rg1, be1, w3, b3)
    h = _to_h4(h, m2, rg2, be2, w4, b4)
    _stats_acc(st_out, h, 256)


def _pass4_body(x, w1, b1, w2, b2, m1, rg1, be1, w3, b3, m2, rg2, be2,
                w4, b4, m3, rg3, be3, w5, b5, w6, b6,
                d12, d13, d23, t12o, t13o, t23o):
    xx = x[...]
    h = _to_h2(xx, w1, b1, w2, b2)
    h = _to_h3(h, m1, rg1, be1, w3, b3)
    h = _to_h4(h, m2, rg2, be2, w4, b4)
    hn = _bn(h, m3[...], rg3[...], be3[...])
    h = _relu(_dot(hn, w5[...]) + b5[...])
    d = _dot(h, w6[...]) + b6[...]
    d12[...] = d[:, 0:8]
    d13[...] = d[:, 8:16]
    d23[...] = d[:, 16:24]
    t12o[...] = xx[:, 0:8] - d[:, 0:8]
    t13o[...] = xx[:, 8:16] - d[:, 8:16]
    t23o[...] = xx[:, 16:24] - d[:, 16:24]


def _row_spec(width):
    return pl.BlockSpec((RP, width), lambda i: (i, 0))


def _wspec():
    return pl.BlockSpec(index_map=lambda i: (0, 0))


_seq = pltpu.CompilerParams(dimension_semantics=("arbitrary",))


def kernel(edge_costs, t12_costs, t13_costs, t23_costs, tri_corr_12,
           tri_corr_13, tri_corr_23, edge_counter, W1, b1, W2, b2, W3, b3,
           W4, b4, W5, b5, W6, b6, g1, be1, g2, be2, g3, be3):
    ec2 = edge_costs.reshape(3125, 512)
    ct2 = edge_counter.reshape(3125, 512)
    q2, ecm2 = pl.pallas_call(
        _prep_body,
        out_shape=[jax.ShapeDtypeStruct((3125, 512), _f32)] * 2,
    )(ec2, ct2)

    g12, g13, g23 = _sc_gather(
        q2.reshape(E), tri_corr_12, tri_corr_13, tri_corr_23)

    eye8 = jnp.eye(8, dtype=_f32)
    ones8 = jnp.ones((8,), _f32)

    def pw(W):
        return jnp.kron(W.T.astype(_f32), eye8)

    def pv(v):
        return jnp.kron(v.astype(_f32), ones8)

    W1p, W2p, W3p, W4p, W5p, W6p = (
        pw(W1).astype(_bf16), pw(W2).astype(_bf16), pw(W3).astype(_bf16),
        pw(W4).astype(_bf16), pw(W5).astype(_bf16), pw(W6).astype(_bf16))
    b1p, b2p, b3p, b4p, b5p, b6p = (
        pv(b1)[None], pv(b2)[None], pv(b3)[None], pv(b4)[None],
        pv(b5)[None], pv(b6)[None])

    def bn_params(stats, feats, g, be):
        s = stats[0].reshape(feats, 8).sum(1)
        sq = stats[1].reshape(feats, 8).sum(1)
        m = s / N
        v = sq / N - m * m
        rg = g / jnp.sqrt(v + 1e-5)
        return pv(m)[None], pv(rg)[None], pv(be)[None]

    gs = [a.reshape(NP, 8) for a in (g12, g13, g23)]
    ts = [a.reshape(NP, 8) for a in (t12_costs, t13_costs, t23_costs)]

    x_p, st1 = pl.pallas_call(
        _pass1_body,
        grid=(GRID,),
        in_specs=[_row_spec(8)] * 6 + [_wspec()] * 4,
        out_specs=[_row_spec(24), pl.BlockSpec((8, 256), lambda i: (0, 0))],
        out_shape=[jax.ShapeDtypeStruct((NP, 24), _f32),
                   jax.ShapeDtypeStruct((8, 256), _f32)],
        compiler_params=_seq,
    )(*gs, *ts, W1p, b1p, W2p, b2p)

    m1p, rg1p, be1p = bn_params(st1, 32, g1, be1)

    st2 = pl.pallas_call(
        _pass2_body,
        grid=(GRID,),
        in_specs=[_row_spec(24)] + [_wspec()] * 9,
        out_specs=pl.BlockSpec((8, 512), lambda i: (0, 0)),
        out_shape=jax.ShapeDtypeStruct((8, 512), _f32),
        compiler_params=_seq,
    )(x_p, W1p, b1p, W2p, b2p, m1p, rg1p, be1p, W3p, b3p)

    m2p, rg2p, be2p = bn_params(st2, 64, g2, be2)

    st3 = pl.pallas_call(
        _pass3_body,
        grid=(GRID,),
        in_specs=[_row_spec(24)] + [_wspec()] * 14,
        out_specs=pl.BlockSpec((8, 256), lambda i: (0, 0)),
        out_shape=jax.ShapeDtypeStruct((8, 256), _f32),
        compiler_params=_seq,
    )(x_p, W1p, b1p, W2p, b2p, m1p, rg1p, be1p, W3p, b3p,
      m2p, rg2p, be2p, W4p, b4p)

    m3p, rg3p, be3p = bn_params(st3, 32, g3, be3)

    d12, d13, d23, t12o, t13o, t23o = pl.pallas_call(
        _pass4_body,
        grid=(GRID,),
        in_specs=[_row_spec(24)] + [_wspec()] * 21,
        out_specs=[_row_spec(8)] * 6,
        out_shape=[jax.ShapeDtypeStruct((NP, 8), _f32)] * 6,
        compiler_params=_seq,
    )(x_p, W1p, b1p, W2p, b2p, m1p, rg1p, be1p, W3p, b3p,
      m2p, rg2p, be2p, W4p, b4p, m3p, rg3p, be3p, W5p, b5p, W6p, b6p)

    zeros = jnp.zeros((E,), _f32)
    parts = _sc_scatter(
        d12.reshape(N), d13.reshape(N), d23.reshape(N),
        tri_corr_12, tri_corr_13, tri_corr_23, zeros)

    ec_out = pl.pallas_call(
        _merge_body,
        out_shape=jax.ShapeDtypeStruct((3125, 512), _f32),
    )(parts[:E].reshape(3125, 512), parts[E:].reshape(3125, 512), ecm2)

    return (ec_out.reshape(E), t12o.reshape(N), t13o.reshape(N),
            t23o.reshape(N))


# speedup vs baseline: 9.9175x; 1.5058x over previous
import functools

import jax
import jax.numpy as jnp
from jax import lax
from jax.experimental import pallas as pl
from jax.experimental.pallas import tpu as pltpu
from jax.experimental.pallas import tpu_sc as plsc

E = 1_600_000
N = 1_600_000
NP = N // 8
R = 12_800
RP = R // 8
GRID = N // R
RW = R // 128
NW2 = N // 128

NC = 2
NS = 16
NW = NC * NS
PW = N // NW
CH = 10_000
SEG = E // NS

_f32 = jnp.float32


def _make_sc_gather():
    mesh = plsc.VectorSubcoreMesh(
        core_axis_name="c", subcore_axis_name="s", num_cores=NC,
        num_subcores=NS)

    @functools.partial(
        pl.kernel, mesh=mesh,
        out_type=[jax.ShapeDtypeStruct((N,), _f32) for _ in range(3)],
        scratch_types=[
            pltpu.VMEM((CH,), jnp.int32),
            pltpu.VMEM((CH,), _f32),
            pltpu.SemaphoreType.DMA,
        ],
    )
    def gather_k(q_hbm, i12, i13, i23, g12, g13, g23, idx_v, val_v, sem):
        wid = lax.axis_index("s") * NC + lax.axis_index("c")
        base = pl.multiple_of(wid * PW, 8)
        for ih, gh in ((i12, g12), (i13, g13), (i23, g23)):
            for c in range(PW // CH):
                off = pl.multiple_of(base + c * CH, 8)
                pltpu.sync_copy(ih.at[pl.ds(off, CH)], idx_v)
                pltpu.async_copy(q_hbm.at[idx_v], val_v, sem).wait()
                pltpu.sync_copy(val_v, gh.at[pl.ds(off, CH)])

    return gather_k


def _make_sc_scatter():
    mesh = plsc.VectorSubcoreMesh(
        core_axis_name="c", subcore_axis_name="s", num_cores=NC,
        num_subcores=NS)

    @functools.partial(
        pl.kernel, mesh=mesh,
        out_type=jax.ShapeDtypeStruct((NC * E,), _f32),
        scratch_types=[
            pltpu.VMEM((CH,), jnp.int32),
            pltpu.VMEM((CH,), _f32),
            pltpu.VMEM_SHARED((E,), _f32),
        ],
    )
    def scatter_k(d12, d13, d23, i12, i13, i23, zeros_hbm, out,
                  idx_v, val_v, acc):
        cid = lax.axis_index("c")
        sid = lax.axis_index("s")
        wid = sid * NC + cid
        soff = pl.multiple_of(sid * SEG, 8)
        for c in range(SEG // CH):
            zoff = pl.multiple_of(soff + c * CH, 8)
            pltpu.sync_copy(zeros_hbm.at[pl.ds(zoff, CH)], val_v)
            pltpu.sync_copy(val_v, acc.at[pl.ds(zoff, CH)])
        plsc.subcore_barrier()
        base = pl.multiple_of(wid * PW, 8)
        for dh, ih in ((d12, i12), (d13, i13), (d23, i23)):
            for c in range(PW // CH):
                off = pl.multiple_of(base + c * CH, 8)
                pltpu.sync_copy(ih.at[pl.ds(off, CH)], idx_v)
                pltpu.sync_copy(dh.at[pl.ds(off, CH)], val_v)
                pltpu.sync_copy(val_v, acc.at[idx_v], add=True)
        plsc.subcore_barrier()
        for c in range(SEG // CH):
            aoff = pl.multiple_of(soff + c * CH, 8)
            ooff = pl.multiple_of(cid * E + soff + c * CH, 8)
            pltpu.sync_copy(acc.at[pl.ds(aoff, CH)], val_v)
            pltpu.sync_copy(val_v, out.at[pl.ds(ooff, CH)])

    return scatter_k


_sc_gather = _make_sc_gather()
_sc_scatter = _make_sc_scatter()


def _relu(x):
    return jnp.maximum(x, 0.0)


def _prep_body(ec_ref, ct_ref, q_ref, ecm_ref):
    ec = ec_ref[...]
    ct = ct_ref[...]
    q_ref[...] = ec / ct
    ecm_ref[...] = jnp.where(ct > 0, jnp.zeros_like(ec), ec)


def _merge_body(p0_ref, p1_ref, ecm_ref, out_ref):
    out_ref[...] = p0_ref[...] + p1_ref[...] + ecm_ref[...]


def _stats_acc(st_ref, h, width):
    s = jnp.sum(h, axis=0)
    sq = jnp.sum(h * h, axis=0)
    acc = jnp.concatenate(
        [s[None], sq[None], jnp.zeros((6, width), _f32)], axis=0)

    @pl.when(pl.program_id(0) == 0)
    def _():
        st_ref[...] = acc

    @pl.when(pl.program_id(0) > 0)
    def _():
        st_ref[...] += acc


_bf16 = jnp.bfloat16


def _dot(a, b):
    return jnp.dot(a.astype(_bf16), b, preferred_element_type=_f32)


def _bn(h, m, rg, be):
    return (h - m) * rg + be


def _to_h2(x, w1, b1, w2, b2):
    h = _relu(_dot(x, w1[...]) + b1[...])
    return _relu(_dot(h, w2[...]) + b2[...])


def _to_h3(h2, m1, rg1, be1, w3, b3):
    hn = _bn(h2, m1[...], rg1[...], be1[...])
    return _relu(_dot(hn, w3[...]) + b3[...])


def _to_h4(h3, m2, rg2, be2, w4, b4):
    hn = _bn(h3, m2[...], rg2[...], be2[...])
    return _relu(_dot(hn, w4[...]) + b4[...])


def _pass1_body(g12, g13, g23, t12, t13, t23, w1, b1, w2, b2, x_out, st_out):
    xc = jnp.concatenate(
        [g12[0] + t12[0], g13[0] + t13[0], g23[0] + t23[0]], axis=0)
    x = jnp.transpose(xc)
    x_out[...] = x
    h = _to_h2(x, w1, b1, w2, b2)
    _stats_acc(st_out, h, 256)


def _pass2_body(x, w1, b1, w2, b2, m1, rg1, be1, w3, b3, st_out):
    h = _to_h2(x[...], w1, b1, w2, b2)
    h = _to_h3(h, m1, rg1, be1, w3, b3)
    _stats_acc(st_out, h, 512)


def _pass3_body(x, w1, b1, w2, b2, m1, rg1, be1, w3, b3, m2, rg2, be2,
                w4, b4, st_out):
    h = _to_h2(x[...], w1, b1, w2, b2)
    h = _to_h3(h, m1, rg1, be1, w3, b3)
    h = _to_h4(h, m2, rg2, be2, w4, b4)
    _stats_acc(st_out, h, 256)


def _pass4_body(x, w1, b1, w2, b2, m1, rg1, be1, w3, b3, m2, rg2, be2,
                w4, b4, m3, rg3, be3, w5, b5, w6, b6,
                d12, d13, d23, t12o, t13o, t23o):
    xx = x[...]
    h = _to_h2(xx, w1, b1, w2, b2)
    h = _to_h3(h, m1, rg1, be1, w3, b3)
    h = _to_h4(h, m2, rg2, be2, w4, b4)
    hn = _bn(h, m3[...], rg3[...], be3[...])
    h = _relu(_dot(hn, w5[...]) + b5[...])
    d = _dot(h, w6[...]) + b6[...]
    dt = jnp.transpose(jnp.concatenate([d, xx - d], axis=1))
    d12[...] = dt[0:8][None]
    d13[...] = dt[8:16][None]
    d23[...] = dt[16:24][None]
    t12o[...] = dt[24:32][None]
    t13o[...] = dt[32:40][None]
    t23o[...] = dt[40:48][None]


def _row_spec(width):
    return pl.BlockSpec((RP, width), lambda i: (i, 0))


def _wspec():
    return pl.BlockSpec(index_map=lambda i: (0, 0))


_seq = pltpu.CompilerParams(dimension_semantics=("arbitrary",))


def kernel(edge_costs, t12_costs, t13_costs, t23_costs, tri_corr_12,
           tri_corr_13, tri_corr_23, edge_counter, W1, b1, W2, b2, W3, b3,
           W4, b4, W5, b5, W6, b6, g1, be1, g2, be2, g3, be3):
    ec2 = edge_costs.reshape(3125, 512)
    ct2 = edge_counter.reshape(3125, 512)
    q2, ecm2 = pl.pallas_call(
        _prep_body,
        out_shape=[jax.ShapeDtypeStruct((3125, 512), _f32)] * 2,
    )(ec2, ct2)

    g12, g13, g23 = _sc_gather(
        q2.reshape(E), tri_corr_12, tri_corr_13, tri_corr_23)

    eye8 = jnp.eye(8, dtype=_f32)
    ones8 = jnp.ones((8,), _f32)

    def pw(W):
        return jnp.kron(W.T.astype(_f32), eye8)

    def pv(v):
        return jnp.kron(v.astype(_f32), ones8)

    W1p, W2p, W3p, W4p, W5p, W6p = (
        pw(W1).astype(_bf16), pw(W2).astype(_bf16), pw(W3).astype(_bf16),
        pw(W4).astype(_bf16), pw(W5).astype(_bf16), pw(W6).astype(_bf16))
    b1p, b2p, b3p, b4p, b5p, b6p = (
        pv(b1)[None], pv(b2)[None], pv(b3)[None], pv(b4)[None],
        pv(b5)[None], pv(b6)[None])

    def bn_params(stats, feats, g, be):
        s = stats[0].reshape(feats, 8).sum(1)
        sq = stats[1].reshape(feats, 8).sum(1)
        m = s / N
        v = sq / N - m * m
        rg = g / jnp.sqrt(v + 1e-5)
        return pv(m)[None], pv(rg)[None], pv(be)[None]

    gs = [a.reshape(GRID, 8, RP) for a in (g12, g13, g23)]
    ts = [a.reshape(GRID, 8, RP) for a in (t12_costs, t13_costs, t23_costs)]
    _flat_spec = pl.BlockSpec((1, 8, RP), lambda i: (i, 0, 0))

    x_p, st1 = pl.pallas_call(
        _pass1_body,
        grid=(GRID,),
        in_specs=[_flat_spec] * 6 + [_wspec()] * 4,
        out_specs=[_row_spec(24), pl.BlockSpec((8, 256), lambda i: (0, 0))],
        out_shape=[jax.ShapeDtypeStruct((NP, 24), _f32),
                   jax.ShapeDtypeStruct((8, 256), _f32)],
        compiler_params=_seq,
    )(*gs, *ts, W1p, b1p, W2p, b2p)

    m1p, rg1p, be1p = bn_params(st1, 32, g1, be1)

    st2 = pl.pallas_call(
        _pass2_body,
        grid=(GRID,),
        in_specs=[_row_spec(24)] + [_wspec()] * 9,
        out_specs=pl.BlockSpec((8, 512), lambda i: (0, 0)),
        out_shape=jax.ShapeDtypeStruct((8, 512), _f32),
        compiler_params=_seq,
    )(x_p, W1p, b1p, W2p, b2p, m1p, rg1p, be1p, W3p, b3p)

    m2p, rg2p, be2p = bn_params(st2, 64, g2, be2)

    st3 = pl.pallas_call(
        _pass3_body,
        grid=(GRID,),
        in_specs=[_row_spec(24)] + [_wspec()] * 14,
        out_specs=pl.BlockSpec((8, 256), lambda i: (0, 0)),
        out_shape=jax.ShapeDtypeStruct((8, 256), _f32),
        compiler_params=_seq,
    )(x_p, W1p, b1p, W2p, b2p, m1p, rg1p, be1p, W3p, b3p,
      m2p, rg2p, be2p, W4p, b4p)

    m3p, rg3p, be3p = bn_params(st3, 32, g3, be3)

    d12, d13, d23, t12o, t13o, t23o = pl.pallas_call(
        _pass4_body,
        grid=(GRID,),
        in_specs=[_row_spec(24)] + [_wspec()] * 21,
        out_specs=[_flat_spec] * 6,
        out_shape=[jax.ShapeDtypeStruct((GRID, 8, RP), _f32)] * 6,
        compiler_params=_seq,
    )(x_p, W1p, b1p, W2p, b2p, m1p, rg1p, be1p, W3p, b3p,
      m2p, rg2p, be2p, W4p, b4p, m3p, rg3p, be3p, W5p, b5p, W6p, b6p)

    zeros = jnp.zeros((E,), _f32)
    parts = _sc_scatter(
        d12.reshape(N), d13.reshape(N), d23.reshape(N),
        tri_corr_12, tri_corr_13, tri_corr_23, zeros)

    ec_out = pl.pallas_call(
        _merge_body,
        out_shape=jax.ShapeDtypeStruct((3125, 512), _f32),
    )(parts[:E].reshape(3125, 512), parts[E:].reshape(3125, 512), ecm2)

    return (ec_out.reshape(E), t12o.reshape(N), t13o.reshape(N),
            t23o.reshape(N))
